# baseline (speedup 1.0000x reference)
import jax
import jax.numpy as jnp
from jax import lax
from jax.experimental import pallas as pl
from jax.experimental.pallas import tpu as pltpu
from jax.experimental.pallas import tpu_sc as plsc
from jax._src import core as _jax_core
from jax._src.pallas.mosaic import core as _tpu_core
from jax._src.pallas.mosaic import sc_lowering as _sc_lowering
from jax.experimental.mosaic.dialects import tpu as _tpu_dialect

_lane_gather_p = _jax_core.Primitive("learned_pos_emb_lane_gather")
_lane_gather_p.def_abstract_eval(
    lambda x, i: _jax_core.ShapedArray(x.shape, x.dtype))


@_sc_lowering.register_lowering_rule(
    _lane_gather_p, kernel_types=[_tpu_core.CoreType.SC_VECTOR_SUBCORE])
def _lane_gather_rule(ctx, x, idx):
    del ctx
    return _tpu_dialect.dynamic_gather(x, idx, dimensions=[0])


_NUM_EMB = 4098
_DIM = 1024
_BATCH = 4
_SEQ = 4096

_NC = 2
_NS = 16
_L = 16
_NW = _NC * _NS

_TOKENS = _BATCH * _SEQ
_COLS_W = _SEQ // _NW
_CH = 16
_NCH = _COLS_W // _CH
_NB = 4
_PRE = _NB - 1


def _all_sum(v):
    lane = lax.iota(jnp.int32, _L)
    for k in (8, 4, 2, 1):
        v = v + _lane_gather_p.bind(v, lane ^ k)
    return v


def _body(ids_hbm, table_hbm, out_hbm, ids_v, pos_v, posl_v, g0, g1, g2,
          g3, gsem0, gsem1, gsem2, gsem3, *ssems_flat):
    wid = lax.axis_index("s") * _NC + lax.axis_index("c")
    col0 = wid * _COLS_W

    gbufs = (g0, g1, g2, g3)
    gsems = (gsem0, gsem1, gsem2, gsem3)
    ssems = tuple(ssems_flat[p * _BATCH:(p + 1) * _BATCH]
                  for p in range(_NB))

    lane = lax.iota(jnp.int32, _L)

    def write_pos(c):
        for i in range(_CH // _L):
            pos_v[c, pl.ds(i * _L, _L)] = (
                col0 + c * _CH + i * _L + 1) + lane

    def gissue(c, p):
        pltpu.async_copy(table_hbm.at[pos_v.at[c]], gbufs[p], gsems[p])

    def gwait(p):
        pltpu.make_async_copy(table_hbm.at[pl.ds(0, _CH)], gbufs[p],
                              gsems[p]).wait()

    def sissue(c, p, b):
        pltpu.async_copy(
            gbufs[p], out_hbm.at[pl.ds(b * _SEQ + col0 + c * _CH, _CH)],
            ssems[p][b])

    def swait(p, b):
        pltpu.make_async_copy(gbufs[p], out_hbm.at[pl.ds(0, _CH)],
                              ssems[p][b]).wait()

    for c in range(_PRE):
        write_pos(c)
        gissue(c, c)
    for b in range(_BATCH):
        pltpu.sync_copy(ids_hbm.at[pl.ds(b * _SEQ + col0, _COLS_W)],
                        ids_v.at[b])
    for c in range(_PRE, _NCH):
        write_pos(c)

    pending = [False] * _NB
    for c in range(_NCH):
        p = c % _NB
        gwait(p)
        for b in range(_BATCH):
            sissue(c, p, b)
        pending[p] = True
        n = c + _PRE
        if n < _NCH:
            q = n % _NB
            if pending[q]:
                for b in range(_BATCH):
                    swait(q, b)
                pending[q] = False
            gissue(n, q)
    for p in range(_NB):
        if pending[p]:
            for b in range(_BATCH):
                swait(p, b)

    for b in range(_BATCH):
        for g in range(_COLS_W // _L):
            grp = ids_v[b, pl.ds(g * _L, _L)]
            nz = _all_sum(jnp.where(grp == 0, 1, 0).astype(jnp.int32))

            @pl.when(nz[0] > 0)
            def _(b=b, g=g, grp=grp):
                col = col0 + g * _L
                posl_v[0, :] = jnp.where(grp == 0, 0, col + 1 + lane)
                fix = pltpu.async_copy(
                    table_hbm.at[posl_v.at[0]], g0.at[pl.ds(0, _L)], gsem0)
                fix.wait()
                pltpu.sync_copy(
                    g0.at[pl.ds(0, _L)],
                    out_hbm.at[pl.ds(b * _SEQ + col, _L)])


@jax.jit
def _lookup(ids_flat, table):
    mesh = plsc.VectorSubcoreMesh(
        core_axis_name="c", subcore_axis_name="s",
        num_cores=_NC, num_subcores=_NS)
    fn = pl.kernel(
        _body,
        out_type=jax.ShapeDtypeStruct((_TOKENS, _DIM), jnp.float32),
        mesh=mesh,
        scratch_types=[
            pltpu.VMEM((_BATCH, _COLS_W), jnp.int32),
            pltpu.VMEM((_NCH, _CH), jnp.int32),
            pltpu.VMEM((1, _L), jnp.int32),
            pltpu.VMEM((_CH, _DIM), jnp.float32),
            pltpu.VMEM((_CH, _DIM), jnp.float32),
            pltpu.VMEM((_CH, _DIM), jnp.float32),
            pltpu.VMEM((_CH, _DIM), jnp.float32),
        ] + [pltpu.SemaphoreType.DMA] * (_NB + _NB * _BATCH),
    )
    return fn(ids_flat, table)


def kernel(input, embeddings):
    ids_flat = input.astype(jnp.int32).reshape(_TOKENS)
    out = _lookup(ids_flat, embeddings)
    return out.reshape(_BATCH, _SEQ, _DIM)


# speedup vs baseline: 1.0036x; 1.0036x over previous
import jax
import jax.numpy as jnp
from jax import lax
from jax.experimental import pallas as pl
from jax.experimental.pallas import tpu as pltpu
from jax.experimental.pallas import tpu_sc as plsc
from jax._src import core as _jax_core
from jax._src.pallas.mosaic import core as _tpu_core
from jax._src.pallas.mosaic import sc_lowering as _sc_lowering
from jax.experimental.mosaic.dialects import tpu as _tpu_dialect

_lane_gather_p = _jax_core.Primitive("learned_pos_emb_lane_gather")
_lane_gather_p.def_abstract_eval(
    lambda x, i: _jax_core.ShapedArray(x.shape, x.dtype))


@_sc_lowering.register_lowering_rule(
    _lane_gather_p, kernel_types=[_tpu_core.CoreType.SC_VECTOR_SUBCORE])
def _lane_gather_rule(ctx, x, idx):
    del ctx
    return _tpu_dialect.dynamic_gather(x, idx, dimensions=[0])


_NUM_EMB = 4098
_DIM = 1024
_BATCH = 4
_SEQ = 4096

_NC = 2
_NS = 16
_L = 16
_NW = _NC * _NS

_TOKENS = _BATCH * _SEQ
_COLS_W = _SEQ // _NW
_CH = 32
_NCH = _COLS_W // _CH
_NB = 3


def _all_sum(v):
    lane = lax.iota(jnp.int32, _L)
    for k in (8, 4, 2, 1):
        v = v + _lane_gather_p.bind(v, lane ^ k)
    return v


def _body(ids_hbm, table_hbm, out_hbm, ids_v, pos_v, posl_v, g0, g1, g2,
          gsem0, gsem1, gsem2, *ssems_flat):
    wid = lax.axis_index("s") * _NC + lax.axis_index("c")
    col0 = wid * _COLS_W

    for b in range(_BATCH):
        pltpu.sync_copy(ids_hbm.at[pl.ds(b * _SEQ + col0, _COLS_W)],
                        ids_v.at[b])

    lane = lax.iota(jnp.int32, _L)
    for c in range(_NCH):
        for i in range(_CH // _L):
            pos_v[c, pl.ds(i * _L, _L)] = (
                col0 + c * _CH + i * _L + 1) + lane

    gbufs = (g0, g1, g2)
    gsems = (gsem0, gsem1, gsem2)
    ssems = tuple(ssems_flat[p * _BATCH:(p + 1) * _BATCH]
                  for p in range(_NB))

    def gissue(c, p):
        pltpu.async_copy(table_hbm.at[pos_v.at[c]], gbufs[p], gsems[p])

    def gwait(p):
        pltpu.make_async_copy(table_hbm.at[pl.ds(0, _CH)], gbufs[p],
                              gsems[p]).wait()

    def sissue(c, p, b):
        pltpu.async_copy(
            gbufs[p], out_hbm.at[pl.ds(b * _SEQ + col0 + c * _CH, _CH)],
            ssems[p][b])

    def swait(p, b):
        pltpu.make_async_copy(gbufs[p], out_hbm.at[pl.ds(0, _CH)],
                              ssems[p][b]).wait()

    gissue(0, 0)
    gissue(1, 1)
    pending = [False] * _NB
    for c in range(_NCH):
        p = c % _NB
        gwait(p)
        for b in range(_BATCH):
            sissue(c, p, b)
        pending[p] = True
        n = c + 2
        if n < _NCH:
            q = n % _NB
            if pending[q]:
                for b in range(_BATCH):
                    swait(q, b)
                pending[q] = False
            gissue(n, q)
    for p in range(_NB):
        if pending[p]:
            for b in range(_BATCH):
                swait(p, b)

    for b in range(_BATCH):
        for g in range(_COLS_W // _L):
            grp = ids_v[b, pl.ds(g * _L, _L)]
            nz = _all_sum(jnp.where(grp == 0, 1, 0).astype(jnp.int32))

            @pl.when(nz[0] > 0)
            def _(b=b, g=g, grp=grp):
                col = col0 + g * _L
                posl_v[0, :] = jnp.where(grp == 0, 0, col + 1 + lane)
                fix = pltpu.async_copy(
                    table_hbm.at[posl_v.at[0]], g0.at[pl.ds(0, _L)], gsem0)
                fix.wait()
                pltpu.sync_copy(
                    g0.at[pl.ds(0, _L)],
                    out_hbm.at[pl.ds(b * _SEQ + col, _L)])


@jax.jit
def _lookup(ids_flat, table):
    mesh = plsc.VectorSubcoreMesh(
        core_axis_name="c", subcore_axis_name="s",
        num_cores=_NC, num_subcores=_NS)
    fn = pl.kernel(
        _body,
        out_type=jax.ShapeDtypeStruct((_TOKENS, _DIM), jnp.float32),
        mesh=mesh,
        scratch_types=[
            pltpu.VMEM((_BATCH, _COLS_W), jnp.int32),
            pltpu.VMEM((_NCH, _CH), jnp.int32),
            pltpu.VMEM((1, _L), jnp.int32),
            pltpu.VMEM((_CH, _DIM), jnp.float32),
            pltpu.VMEM((_CH, _DIM), jnp.float32),
            pltpu.VMEM((_CH, _DIM), jnp.float32),
        ] + [pltpu.SemaphoreType.DMA] * (3 + _NB * _BATCH),
    )
    return fn(ids_flat, table)


def kernel(input, embeddings):
    ids_flat = input.astype(jnp.int32).reshape(_TOKENS)
    out = _lookup(ids_flat, embeddings)
    return out.reshape(_BATCH, _SEQ, _DIM)
